# baseline (speedup 1.0000x reference)
import functools

import jax
import jax.numpy as jnp
from jax import lax
from jax.experimental import pallas as pl
from jax.experimental.pallas import tpu as pltpu
from jax.experimental.pallas import tpu_sc as plsc

_D = 64
_K = 1024
_M = 2048
_NC = 2
_NS = 16
_NW = _NC * _NS
_CH = 128


def _conv_xla(x, w, b):
    y = lax.conv_general_dilated(x, w, (1, 1), 'VALID',
                                 dimension_numbers=('NCHW', 'OIHW', 'NCHW'))
    return y + b[None, :, None, None]


def _deconv_xla(x, w, b):
    y = lax.conv_general_dilated(x, jnp.flip(w, (2, 3)), (1, 1),
                                 [(4, 4), (4, 4)],
                                 dimension_numbers=('NCHW', 'OIHW', 'NCHW'))
    return y + b[None, :, None, None]


def _vq_argmin_body(z_ref, e_ref, idx_ref):
    z = z_ref[...]
    e = e_ref[...]
    scores = lax.dot_general(z, e, (((1,), (1,)), ((), ())),
                             preferred_element_type=jnp.float32)
    z2 = jnp.sum(z * z, axis=1, keepdims=True)
    e2 = jnp.sum(e * e, axis=1)[None, :]
    d = z2 - 2.0 * scores + e2
    m = jnp.min(d, axis=1, keepdims=True)
    iota = lax.broadcasted_iota(jnp.int32, d.shape, 1)
    idx = jnp.min(jnp.where(d == m, iota, _K), axis=1)
    idx_ref[...] = idx[None, None, :]


def _vq_argmin(zt_pad, emb):
    n = zt_pad.shape[0]
    nblk = n // _M
    out = pl.pallas_call(
        _vq_argmin_body,
        grid=(nblk,),
        in_specs=[
            pl.BlockSpec((_M, _D), lambda i: (i, 0)),
            pl.BlockSpec((_K, _D), lambda i: (0, 0)),
        ],
        out_specs=pl.BlockSpec((1, 1, _M), lambda i: (i, 0, 0)),
        out_shape=jax.ShapeDtypeStruct((nblk, 1, _M), jnp.int32),
    )(zt_pad, emb)
    return out.reshape(n)


def _sc_gather(idx3d, emb):
    rpw = idx3d.shape[1]
    n = _NW * rpw * _CH
    mesh = plsc.VectorSubcoreMesh(core_axis_name="c", subcore_axis_name="s")

    @functools.partial(
        pl.kernel, mesh=mesh,
        compiler_params=pltpu.CompilerParams(use_tc_tiling_on_sc=False),
        out_type=jax.ShapeDtypeStruct((n, _D), jnp.float32),
        scratch_types=[
            pltpu.VMEM((rpw, _CH), jnp.int32),
            pltpu.VMEM((_CH, _D), jnp.float32),
            pltpu.SemaphoreType.DMA,
        ],
    )
    def k(idx_hbm, emb_hbm, out_hbm, idx_v, rows_v, sem):
        wid = lax.axis_index("s") * _NC + lax.axis_index("c")
        pltpu.sync_copy(idx_hbm.at[wid], idx_v)
        for j in range(rpw):
            pltpu.async_copy(emb_hbm.at[idx_v.at[j]], rows_v, sem).wait()
            pltpu.sync_copy(rows_v,
                            out_hbm.at[pl.ds((wid * rpw + j) * _CH, _CH)])

    return k(idx3d, emb)


def kernel(x, We1, be1, We2, be2, We3, be3, Wd1, bd1, Wd2, bd2, Wd3, bd3,
           embedding):
    h = jax.nn.selu(_conv_xla(x, We1, be1))
    h = jax.nn.selu(_conv_xla(h, We2, be2))
    z = _conv_xla(h, We3, be3)
    b, d, hh, ww = z.shape
    zt = jnp.transpose(z, (0, 2, 3, 1)).reshape(b * hh * ww, d)
    n0 = zt.shape[0]
    n = -(-n0 // (_NW * _CH)) * (_NW * _CH)
    zt_pad = jnp.pad(zt, ((0, n - n0), (0, 0)))
    idx_flat = _vq_argmin(zt_pad, embedding)
    q_flat = _sc_gather(idx_flat.reshape(_NW, n // (_NW * _CH), _CH),
                        embedding)
    idx = idx_flat[:n0].reshape(b, hh, ww)
    q = q_flat[:n0].reshape(b, hh, ww, d).transpose(0, 3, 1, 2)
    g = jax.nn.selu(_deconv_xla(q, Wd1, bd1))
    g = jax.nn.selu(_deconv_xla(g, Wd2, bd2))
    out = _deconv_xla(g, Wd3, bd3)
    return out, idx


# speedup vs baseline: 1.0151x; 1.0151x over previous
import functools

import jax
import jax.numpy as jnp
from jax import lax
from jax.experimental import pallas as pl
from jax.experimental.pallas import tpu as pltpu
from jax.experimental.pallas import tpu_sc as plsc

_D = 64
_K = 1024
_M = 2048
_NC = 2
_NS = 16
_NW = _NC * _NS
_CH = 128
_WAVE = 11


def _vq_argmin_body(z_ref, e_ref, idx_ref):
    z = z_ref[...]
    e = e_ref[...]
    scores = lax.dot_general(z.astype(jnp.bfloat16), e.astype(jnp.bfloat16),
                             (((1,), (1,)), ((), ())),
                             preferred_element_type=jnp.float32)
    z2 = jnp.sum(z * z, axis=1, keepdims=True)
    e2 = jnp.sum(e * e, axis=1)[None, :]
    d = z2 - 2.0 * scores + e2
    m = jnp.min(d, axis=1, keepdims=True)
    iota = lax.broadcasted_iota(jnp.int32, d.shape, 1)
    idx = jnp.min(jnp.where(d == m, iota, _K - 1), axis=1)
    idx_ref[...] = idx[None, None, :]


def _vq_argmin(zt, emb, nblk):
    return pl.pallas_call(
        _vq_argmin_body,
        grid=(nblk,),
        in_specs=[
            pl.BlockSpec((_M, _D), lambda i: (i, 0)),
            pl.BlockSpec((_K, _D), lambda i: (0, 0)),
        ],
        out_specs=pl.BlockSpec((1, 1, _M), lambda i: (i, 0, 0)),
        out_shape=jax.ShapeDtypeStruct((nblk, 1, _M), jnp.int32),
    )(zt, emb).reshape(nblk * _M)


def _sc_gather(idx3d, emb):
    rpw = idx3d.shape[1]
    n = _NW * rpw * _CH
    nwave = rpw // _WAVE
    mesh = plsc.VectorSubcoreMesh(core_axis_name="c", subcore_axis_name="s")

    @functools.partial(
        pl.kernel, mesh=mesh,
        compiler_params=pltpu.CompilerParams(use_tc_tiling_on_sc=False),
        out_type=jax.ShapeDtypeStruct((n, _D), jnp.float32),
        scratch_types=[
            pltpu.VMEM((rpw, _CH), jnp.int32),
            pltpu.VMEM((_WAVE * _CH, _D), jnp.float32),
            pltpu.SemaphoreType.DMA,
        ],
    )
    def k(idx_hbm, emb_hbm, out_hbm, idx_v, rows_v, sem):
        wid = lax.axis_index("s") * _NC + lax.axis_index("c")
        pltpu.sync_copy(idx_hbm.at[wid], idx_v)
        for g in range(nwave):
            cps = [
                pltpu.async_copy(emb_hbm.at[idx_v.at[g * _WAVE + i]],
                                 rows_v.at[pl.ds(i * _CH, _CH)], sem)
                for i in range(_WAVE)
            ]
            for c in cps:
                c.wait()
            pltpu.sync_copy(
                rows_v,
                out_hbm.at[pl.ds((wid * rpw + g * _WAVE) * _CH, _WAVE * _CH)])

    return k(idx3d, emb)


def kernel(x, We1, be1, We2, be2, We3, be3, Wd1, bd1, Wd2, bd2, Wd3, bd3,
           embedding):
    def conv(v, w, b, dn):
        y = lax.conv_general_dilated(v, w, (1, 1), 'VALID',
                                     dimension_numbers=dn)
        return y + b

    def deconv(v, w, b, dn):
        y = lax.conv_general_dilated(v, jnp.flip(w, (2, 3)), (1, 1),
                                     [(4, 4), (4, 4)], dimension_numbers=dn)
        return y + b

    h = jax.nn.selu(conv(x, We1, be1, ('NCHW', 'OIHW', 'NHWC')))
    h = jax.nn.selu(conv(h, We2, be2, ('NHWC', 'OIHW', 'NHWC')))
    z = conv(h, We3, be3, ('NHWC', 'OIHW', 'NHWC'))
    b, hh, ww, d = z.shape
    n0 = b * hh * ww
    n = -(-n0 // (_NW * _CH * _WAVE)) * (_NW * _CH * _WAVE)
    nblk = n // _M
    idx_flat = _vq_argmin(z.reshape(n0, d), embedding, nblk)
    q_flat = _sc_gather(idx_flat.reshape(_NW, n // (_NW * _CH), _CH),
                        embedding)
    idx = idx_flat[:n0].reshape(b, hh, ww)
    q = q_flat[:n0].reshape(b, hh, ww, d)
    g = jax.nn.selu(deconv(q, Wd1, bd1, ('NHWC', 'OIHW', 'NHWC')))
    g = jax.nn.selu(deconv(g, Wd2, bd2, ('NHWC', 'OIHW', 'NHWC')))
    out = deconv(g, Wd3, bd3, ('NHWC', 'OIHW', 'NHWC'))
    return out.transpose(0, 3, 1, 2), idx


# speedup vs baseline: 1.0233x; 1.0081x over previous
import functools

import jax
import jax.numpy as jnp
from jax import lax
from jax.experimental import pallas as pl
from jax.experimental.pallas import tpu as pltpu
from jax.experimental.pallas import tpu_sc as plsc

_D = 64
_K = 1024
_M = 4096
_NC = 2
_NS = 16
_NW = _NC * _NS
_CH = 128
_WAVE = 11


def _vq_argmin_body(z_ref, eb_ref, e2_ref, idx_ref):
    z = z_ref[...]
    scores = lax.dot_general(z.astype(jnp.bfloat16), eb_ref[...],
                             (((1,), (1,)), ((), ())),
                             preferred_element_type=jnp.float32)
    z2 = jnp.sum(z * z, axis=1, keepdims=True)
    d = z2 - 2.0 * scores + e2_ref[...]
    m = jnp.min(d, axis=1, keepdims=True)
    iota = lax.broadcasted_iota(jnp.int32, d.shape, 1)
    idx = jnp.min(jnp.where(d == m, iota, _K - 1), axis=1)
    idx_ref[...] = idx[None, None, :]


def _vq_argmin(zt, emb, nblk):
    eb = emb.astype(jnp.bfloat16)
    e2 = jnp.sum(emb ** 2, axis=-1)[None, :]
    return pl.pallas_call(
        _vq_argmin_body,
        grid=(nblk,),
        in_specs=[
            pl.BlockSpec((_M, _D), lambda i: (i, 0)),
            pl.BlockSpec((_K, _D), lambda i: (0, 0)),
            pl.BlockSpec((1, _K), lambda i: (0, 0)),
        ],
        out_specs=pl.BlockSpec((1, 1, _M), lambda i: (i, 0, 0)),
        out_shape=jax.ShapeDtypeStruct((nblk, 1, _M), jnp.int32),
    )(zt, eb, e2).reshape(nblk * _M)


def _sc_gather(idx3d, emb):
    rpw = idx3d.shape[1]
    n = _NW * rpw * _CH
    nwave = rpw // _WAVE
    mesh = plsc.VectorSubcoreMesh(core_axis_name="c", subcore_axis_name="s")

    @functools.partial(
        pl.kernel, mesh=mesh,
        compiler_params=pltpu.CompilerParams(use_tc_tiling_on_sc=False),
        out_type=jax.ShapeDtypeStruct((n, _D), jnp.float32),
        scratch_types=[
            pltpu.VMEM((rpw, _CH), jnp.int32),
            pltpu.VMEM((_WAVE * _CH, _D), jnp.float32),
            pltpu.SemaphoreType.DMA,
        ],
    )
    def k(idx_hbm, emb_hbm, out_hbm, idx_v, rows_v, sem):
        wid = lax.axis_index("s") * _NC + lax.axis_index("c")
        pltpu.sync_copy(idx_hbm.at[wid], idx_v)
        for g in range(nwave):
            cps = [
                pltpu.async_copy(emb_hbm.at[idx_v.at[g * _WAVE + i]],
                                 rows_v.at[pl.ds(i * _CH, _CH)], sem)
                for i in range(_WAVE)
            ]
            for c in cps:
                c.wait()
            pltpu.sync_copy(
                rows_v,
                out_hbm.at[pl.ds((wid * rpw + g * _WAVE) * _CH, _WAVE * _CH)])

    return k(idx3d, emb)


def kernel(x, We1, be1, We2, be2, We3, be3, Wd1, bd1, Wd2, bd2, Wd3, bd3,
           embedding):
    def conv(v, w, b, dn):
        y = lax.conv_general_dilated(v, w, (1, 1), 'VALID',
                                     dimension_numbers=dn)
        return y + b

    def deconv(v, w, b, dn):
        y = lax.conv_general_dilated(v, jnp.flip(w, (2, 3)), (1, 1),
                                     [(4, 4), (4, 4)], dimension_numbers=dn)
        return y + b

    h = jax.nn.selu(conv(x, We1, be1, ('NCHW', 'OIHW', 'NHWC')))
    h = jax.nn.selu(conv(h, We2, be2, ('NHWC', 'OIHW', 'NHWC')))
    z = conv(h, We3, be3, ('NHWC', 'OIHW', 'NHWC'))
    b, hh, ww, d = z.shape
    n0 = b * hh * ww
    n = -(-n0 // (_NW * _CH * _WAVE)) * (_NW * _CH * _WAVE)
    nblk = n // _M
    idx_flat = _vq_argmin(z.reshape(n0, d), embedding, nblk)
    q_flat = _sc_gather(idx_flat.reshape(_NW, n // (_NW * _CH), _CH),
                        embedding)
    idx = idx_flat[:n0].reshape(b, hh, ww)
    q = q_flat[:n0].reshape(b, hh, ww, d)
    g = jax.nn.selu(deconv(q, Wd1, bd1, ('NHWC', 'OIHW', 'NHWC')))
    g = jax.nn.selu(deconv(g, Wd2, bd2, ('NHWC', 'OIHW', 'NHWC')))
    out = deconv(g, Wd3, bd3, ('NHWC', 'OIHW', 'NHWC'))
    return out.transpose(0, 3, 1, 2), idx


# speedup vs baseline: 1.1475x; 1.1214x over previous
import functools

import jax
import jax.numpy as jnp
from jax import lax
from jax.experimental import pallas as pl
from jax.experimental.pallas import tpu as pltpu
from jax.experimental.pallas import tpu_sc as plsc

_D = 64
_K = 1024
_M = 4096
_NC = 2
_NS = 16
_NW = _NC * _NS
_CH = 128
_WAVE = 11


_RB = 36


def _conv_body(xa_ref, xb_ref, w_ref, b_ref, o_ref, *, wout, cin, cout, act):
    xa = xa_ref[0]
    xb = xb_ref[0][:4]
    x = jnp.concatenate([xa, xb], axis=0)
    xc = jnp.concatenate([x[:, dx:dx + wout, :] for dx in range(5)], axis=2)
    xcb = xc.astype(jnp.bfloat16).reshape((_RB + 4) * wout, 5 * cin)
    u = lax.dot_general(xcb, w_ref[...], (((1,), (0,)), ((), ())),
                        preferred_element_type=jnp.float32)
    u = u.reshape(_RB + 4, wout, 5 * cout)
    y = u[0:_RB, :, 0:cout]
    for dy in range(1, 5):
        y = y + u[dy:dy + _RB, :, dy * cout:(dy + 1) * cout]
    y = y + b_ref[...][0]
    if act:
        alpha, scale = 1.6732632423543772, 1.0507009873554805
        y = scale * jnp.where(y > 0, y, alpha * (jnp.exp(y) - 1.0))
    o_ref[0] = y


def _pconv(x, w, b, act):
    bb, hin, win, cin = x.shape
    cout = w.shape[0]
    hout, wout = hin - 4, win - 4
    nblk = -(-hout // _RB)
    nin_blk = -(-hin // _RB)
    wmat = jnp.transpose(w, (3, 1, 2, 0)).reshape(5 * cin, 5 * cout)
    wmat = wmat.astype(jnp.bfloat16)
    body = functools.partial(_conv_body, wout=wout, cin=cin, cout=cout,
                             act=act)
    return pl.pallas_call(
        body,
        grid=(bb, nblk),
        in_specs=[
            pl.BlockSpec((1, _RB, win, cin), lambda bi, i: (bi, i, 0, 0)),
            pl.BlockSpec((1, _RB, win, cin),
                         lambda bi, i: (bi, jnp.minimum(i + 1, nin_blk - 1),
                                        0, 0)),
            pl.BlockSpec((5 * cin, 5 * cout), lambda bi, i: (0, 0)),
            pl.BlockSpec((1, cout), lambda bi, i: (0, 0)),
        ],
        out_specs=pl.BlockSpec((1, _RB, wout, cout),
                               lambda bi, i: (bi, i, 0, 0)),
        out_shape=jax.ShapeDtypeStruct((bb, hout, wout, cout), jnp.float32),
    )(x, x, wmat, b.reshape(1, cout))


def _vq_argmin_body(z_ref, eb_ref, e2_ref, idx_ref):
    z = z_ref[...]
    scores = lax.dot_general(z.astype(jnp.bfloat16), eb_ref[...],
                             (((1,), (1,)), ((), ())),
                             preferred_element_type=jnp.float32)
    z2 = jnp.sum(z * z, axis=1, keepdims=True)
    d = z2 - 2.0 * scores + e2_ref[...]
    m = jnp.min(d, axis=1, keepdims=True)
    iota = lax.broadcasted_iota(jnp.int32, d.shape, 1)
    idx = jnp.min(jnp.where(d == m, iota, _K - 1), axis=1)
    idx_ref[...] = idx[None, None, :]


def _vq_argmin(zt, emb, nblk):
    eb = emb.astype(jnp.bfloat16)
    e2 = jnp.sum(emb ** 2, axis=-1)[None, :]
    return pl.pallas_call(
        _vq_argmin_body,
        grid=(nblk,),
        in_specs=[
            pl.BlockSpec((_M, _D), lambda i: (i, 0)),
            pl.BlockSpec((_K, _D), lambda i: (0, 0)),
            pl.BlockSpec((1, _K), lambda i: (0, 0)),
        ],
        out_specs=pl.BlockSpec((1, 1, _M), lambda i: (i, 0, 0)),
        out_shape=jax.ShapeDtypeStruct((nblk, 1, _M), jnp.int32),
    )(zt, eb, e2).reshape(nblk * _M)


def _sc_gather(idx3d, emb):
    rpw = idx3d.shape[1]
    n = _NW * rpw * _CH
    nwave = rpw // _WAVE
    mesh = plsc.VectorSubcoreMesh(core_axis_name="c", subcore_axis_name="s")

    @functools.partial(
        pl.kernel, mesh=mesh,
        compiler_params=pltpu.CompilerParams(use_tc_tiling_on_sc=False),
        out_type=jax.ShapeDtypeStruct((n, _D), jnp.float32),
        scratch_types=[
            pltpu.VMEM((rpw, _CH), jnp.int32),
            pltpu.VMEM((_WAVE * _CH, _D), jnp.float32),
            pltpu.SemaphoreType.DMA,
        ],
    )
    def k(idx_hbm, emb_hbm, out_hbm, idx_v, rows_v, sem):
        wid = lax.axis_index("s") * _NC + lax.axis_index("c")
        pltpu.sync_copy(idx_hbm.at[wid], idx_v)
        for g in range(nwave):
            cps = [
                pltpu.async_copy(emb_hbm.at[idx_v.at[g * _WAVE + i]],
                                 rows_v.at[pl.ds(i * _CH, _CH)], sem)
                for i in range(_WAVE)
            ]
            for c in cps:
                c.wait()
            pltpu.sync_copy(
                rows_v,
                out_hbm.at[pl.ds((wid * rpw + g * _WAVE) * _CH, _WAVE * _CH)])

    return k(idx3d, emb)


def kernel(x, We1, be1, We2, be2, We3, be3, Wd1, bd1, Wd2, bd2, Wd3, bd3,
           embedding):
    def conv(v, w, b, dn):
        y = lax.conv_general_dilated(v, w, (1, 1), 'VALID',
                                     dimension_numbers=dn)
        return y + b

    def deconv(v, w, b, dn):
        y = lax.conv_general_dilated(v, jnp.flip(w, (2, 3)), (1, 1),
                                     [(4, 4), (4, 4)], dimension_numbers=dn)
        return y + b

    def spad(v):
        return jnp.pad(v, ((0, 0), (4, 4), (4, 4), (0, 0)))

    h = jax.nn.selu(conv(x, We1, be1, ('NCHW', 'OIHW', 'NHWC')))
    h = _pconv(h, We2, be2, act=True)
    z = _pconv(h, We3, be3, act=False)
    b, hh, ww, d = z.shape
    n0 = b * hh * ww
    n = -(-n0 // (_NW * _CH * _WAVE)) * (_NW * _CH * _WAVE)
    nblk = n // _M
    idx_flat = _vq_argmin(z.reshape(n0, d), embedding, nblk)
    q_flat = _sc_gather(idx_flat.reshape(_NW, n // (_NW * _CH), _CH),
                        embedding)
    idx = idx_flat[:n0].reshape(b, hh, ww)
    q = q_flat[:n0].reshape(b, hh, ww, d)
    g = _pconv(spad(q), jnp.flip(Wd1, (2, 3)), bd1, act=True)
    g = _pconv(spad(g), jnp.flip(Wd2, (2, 3)), bd2, act=True)
    out = deconv(g, Wd3, bd3, ('NHWC', 'OIHW', 'NHWC'))
    return out.transpose(0, 3, 1, 2), idx
